# baseline (speedup 1.0000x reference)
import functools

import jax
import jax.numpy as jnp
from jax.experimental import pallas as pl
from jax.experimental.pallas import tpu as pltpu

_K = 16


def _knn_lap_kernel(p1t_ref, p1T_ref, p1e_ref, p2t_ref, p2e_ref, out_ref,
                    d_ref, *, tile_m, n):
    i = pl.program_id(1)

    p1T = p1T_ref[0]
    x = p1T[0:1, :]
    y = p1T[1:2, :]
    z = p1T[2:3, :]
    sqj = x * x + y * y + z * z
    mrow = jnp.concatenate([-2.0 * p1T, sqj], axis=0)

    p1t = p1t_ref[0]
    te = jnp.concatenate([p1t, jnp.ones((tile_m, 1), jnp.float32)], axis=1)

    d = jnp.dot(te, mrow, preferred_element_type=jnp.float32)

    rows = jax.lax.broadcasted_iota(jnp.int32, (tile_m, n), 0) + i * tile_m
    cols = jax.lax.broadcasted_iota(jnp.int32, (tile_m, n), 1)
    d_ref[...] = jnp.where(cols == rows, jnp.inf, d)

    dd = d_ref[...]
    m = jnp.full((tile_m, 1), -jnp.inf, jnp.float32)
    for _ in range(_K):
        m = jnp.min(jnp.where(dd > m, dd, jnp.inf), axis=1, keepdims=True)

    adj = jnp.where(dd <= m, 1.0, 0.0)
    s1e = jnp.dot(adj, p1e_ref[0], preferred_element_type=jnp.float32)
    s2e = jnp.dot(adj, p2e_ref[0], preferred_element_type=jnp.float32)
    cnt = s1e[:, 3:4]
    p2t = p2t_ref[0]
    s1 = s1e[:, 0:3]
    s2 = s2e[:, 0:3]
    diff = (s1 - s2) / cnt - (p1t - p2t)
    part = jnp.sum(jnp.abs(diff))

    @pl.when(i == 0)
    def _init():
        out_ref[...] = jnp.zeros_like(out_ref)

    out_ref[...] += part


def kernel(point1, point2):
    B, n, _ = point1.shape
    tile_m = min(2048, n)
    p1T = jnp.transpose(point1, (0, 2, 1))
    ones = jnp.ones((B, n, 1), jnp.float32)
    p1e = jnp.concatenate([point1, ones], axis=2)
    p2e = jnp.concatenate([point2, ones], axis=2)
    grid = (B, n // tile_m)
    kern = functools.partial(_knn_lap_kernel, tile_m=tile_m, n=n)
    partials = pl.pallas_call(
        kern,
        grid=grid,
        in_specs=[
            pl.BlockSpec((1, tile_m, 3), lambda b, i: (b, i, 0)),
            pl.BlockSpec((1, 3, n), lambda b, i: (b, 0, 0)),
            pl.BlockSpec((1, n, 4), lambda b, i: (b, 0, 0)),
            pl.BlockSpec((1, tile_m, 3), lambda b, i: (b, i, 0)),
            pl.BlockSpec((1, n, 4), lambda b, i: (b, 0, 0)),
        ],
        out_specs=pl.BlockSpec((1, 1, 1), lambda b, i: (b, 0, 0)),
        out_shape=jax.ShapeDtypeStruct((B, 1, 1), jnp.float32),
        scratch_shapes=[pltpu.VMEM((tile_m, n), jnp.float32)],
        compiler_params=pltpu.CompilerParams(
            dimension_semantics=("parallel", "arbitrary"),
        ),
    )(point1, p1T, p1e, point2, p2e)
    return jnp.sum(partials) / (B * n * 3)


# speedup vs baseline: 1.2118x; 1.2118x over previous
import functools

import jax
import jax.numpy as jnp
from jax.experimental import pallas as pl
from jax.experimental.pallas import tpu as pltpu

_K = 16


def _knn_lap_kernel(p1t_ref, p1T_ref, p1e_ref, p2t_ref, p2e_ref, out_ref,
                    d_ref, *, tile_m, n):
    i = pl.program_id(1)

    p1T = p1T_ref[0]
    x = p1T[0:1, :]
    y = p1T[1:2, :]
    z = p1T[2:3, :]
    sqj = x * x + y * y + z * z
    mrow = jnp.concatenate([-2.0 * p1T, sqj], axis=0)

    p1t = p1t_ref[0]
    te = jnp.concatenate([p1t, jnp.ones((tile_m, 1), jnp.float32)], axis=1)

    d = jnp.dot(te, mrow, preferred_element_type=jnp.float32)

    rows = jax.lax.broadcasted_iota(jnp.int32, (tile_m, n), 0) + i * tile_m
    cols = jax.lax.broadcasted_iota(jnp.int32, (tile_m, n), 1)
    d_ref[...] = jnp.where(cols == rows, jnp.inf, d)

    dd = d_ref[...]
    m = jnp.min(dd, axis=1, keepdims=True)
    for _ in range(_K - 1):
        m = jnp.min(jnp.where(dd > m, dd, jnp.inf), axis=1, keepdims=True)

    adj = jnp.where(dd <= m, 1.0, 0.0)
    s1e = jnp.dot(adj, p1e_ref[0], preferred_element_type=jnp.float32)
    s2e = jnp.dot(adj, p2e_ref[0], preferred_element_type=jnp.float32)
    cnt = s1e[:, 3:4]
    p2t = p2t_ref[0]
    s1 = s1e[:, 0:3]
    s2 = s2e[:, 0:3]
    diff = (s1 - s2) / cnt - (p1t - p2t)
    part = jnp.sum(jnp.abs(diff))

    @pl.when(i == 0)
    def _init():
        out_ref[...] = jnp.zeros_like(out_ref)

    out_ref[...] += part


def kernel(point1, point2):
    B, n, _ = point1.shape
    tile_m = min(1024, n)
    p1T = jnp.transpose(point1, (0, 2, 1))
    ones = jnp.ones((B, n, 1), jnp.float32)
    p1e = jnp.concatenate([point1, ones], axis=2)
    p2e = jnp.concatenate([point2, ones], axis=2)
    grid = (B, n // tile_m)
    kern = functools.partial(_knn_lap_kernel, tile_m=tile_m, n=n)
    partials = pl.pallas_call(
        kern,
        grid=grid,
        in_specs=[
            pl.BlockSpec((1, tile_m, 3), lambda b, i: (b, i, 0)),
            pl.BlockSpec((1, 3, n), lambda b, i: (b, 0, 0)),
            pl.BlockSpec((1, n, 4), lambda b, i: (b, 0, 0)),
            pl.BlockSpec((1, tile_m, 3), lambda b, i: (b, i, 0)),
            pl.BlockSpec((1, n, 4), lambda b, i: (b, 0, 0)),
        ],
        out_specs=pl.BlockSpec((1, 1, 1), lambda b, i: (b, 0, 0)),
        out_shape=jax.ShapeDtypeStruct((B, 1, 1), jnp.float32),
        scratch_shapes=[pltpu.VMEM((tile_m, n), jnp.float32)],
        compiler_params=pltpu.CompilerParams(
            dimension_semantics=("parallel", "arbitrary"),
        ),
    )(point1, p1T, p1e, point2, p2e)
    return jnp.sum(partials) / (B * n * 3)
